# baseline (speedup 1.0000x reference)
import functools
import math

import jax
import jax.numpy as jnp
from jax import lax
from jax.experimental import pallas as pl
from jax.experimental.pallas import tpu as pltpu
from jax.experimental.pallas import tpu_sc as plsc

VOCAB = 1_000_000
EMB = 32
B = 4096
L = 200
N = B * L

NC = 2
NS = 16
NW = NC * NS

IDX_MINOR = 128
CHUNK_TOK = 1024
CHUNK_IROWS = CHUNK_TOK // IDX_MINOR
CHUNKS_PER_L = B // CHUNK_TOK
N_CHUNKS_TOTAL = N // CHUNK_TOK
CHUNKS_PER_W = N_CHUNKS_TOTAL // NW

SCALE = math.sqrt(float(EMB))

_mesh = plsc.VectorSubcoreMesh(
    core_axis_name="c", subcore_axis_name="s", num_cores=NC, num_subcores=NS
)


@functools.partial(
    pl.kernel,
    out_type=jax.ShapeDtypeStruct((L, EMB // 8, B // 128, 8, 128), jnp.float32),
    mesh=_mesh,
    scratch_types=[
        pltpu.VMEM((CHUNK_IROWS, IDX_MINOR), jnp.int32),
        pltpu.VMEM((CHUNK_TOK, EMB), jnp.float32),
        pltpu.VMEM((EMB // 8, CHUNK_TOK // 128, 8, 128), jnp.float32),
        pltpu.SemaphoreType.DMA,
    ],
    compiler_params=pltpu.CompilerParams(
        use_tc_tiling_on_sc=False, needs_layout_passes=False
    ),
)
def _emb_lookup(idx_hbm, table_hbm, out_hbm, idx_v, rows_v, trans_v, sem):
    wid = lax.axis_index("s") * NC + lax.axis_index("c")
    lane_iota = lax.iota(jnp.int32, 16)

    def chunk_body(ci, carry):
        c = wid * CHUNKS_PER_W + ci
        l = c // CHUNKS_PER_L
        bt0 = (c % CHUNKS_PER_L) * (CHUNK_TOK // 128)
        pltpu.sync_copy(idx_hbm.at[pl.ds(c * CHUNK_IROWS, CHUNK_IROWS)], idx_v)
        for j in range(CHUNK_IROWS):
            pltpu.async_copy(
                table_hbm.at[idx_v.at[j]],
                rows_v.at[pl.ds(j * IDX_MINOR, IDX_MINOR)],
                sem,
            )
        for j in range(CHUNK_IROWS):
            pltpu.make_async_copy(
                table_hbm.at[idx_v.at[j]],
                rows_v.at[pl.ds(j * IDX_MINOR, IDX_MINOR)],
                sem,
            ).wait()

        def trans_body(btl, cc):
            row_base = btl * 128
            for et in range(EMB // 8):
                for es in range(8):
                    e = et * 8 + es
                    col_ids = jnp.full((16,), e, jnp.int32)
                    for g in range(8):
                        row_ids = row_base + g * 16 + lane_iota
                        vals = plsc.load_gather(rows_v, [row_ids, col_ids])
                        trans_v[et, btl, es, pl.ds(g * 16, 16)] = vals * SCALE
            return cc

        lax.fori_loop(0, CHUNK_TOK // 128, trans_body, 0)
        pltpu.sync_copy(
            trans_v,
            out_hbm.at[l, :, pl.ds(bt0, CHUNK_TOK // 128)],
        )
        return carry

    lax.fori_loop(0, CHUNKS_PER_W, chunk_body, 0)


def kernel(tokens, W):
    idx = jnp.reshape(tokens.T.astype(jnp.int32), (N // IDX_MINOR, IDX_MINOR))
    out5 = _emb_lookup(idx, W)
    return jnp.transpose(out5, (2, 4, 0, 1, 3)).reshape(B, L, EMB)


# speedup vs baseline: 1.3602x; 1.3602x over previous
import functools
import math

import jax
import jax.numpy as jnp
from jax import lax
from jax.experimental import pallas as pl
from jax.experimental.pallas import tpu as pltpu
from jax.experimental.pallas import tpu_sc as plsc

VOCAB = 1_000_000
EMB = 32
B = 4096
L = 200
N = B * L

NC = 2
NS = 16
NW = NC * NS

IDX_MINOR = 128
CHUNK_TOK = 1024
CHUNK_IROWS = CHUNK_TOK // IDX_MINOR
CHUNKS_PER_L = B // CHUNK_TOK
N_CHUNKS_TOTAL = N // CHUNK_TOK
CHUNKS_PER_W = N_CHUNKS_TOTAL // NW

SCALE = math.sqrt(float(EMB))

_mesh = plsc.VectorSubcoreMesh(
    core_axis_name="c", subcore_axis_name="s", num_cores=NC, num_subcores=NS
)


@functools.partial(
    pl.kernel,
    out_type=jax.ShapeDtypeStruct((L, EMB // 8, B // 128, 8, 128), jnp.float32),
    mesh=_mesh,
    scratch_types=[
        pltpu.VMEM((CHUNK_IROWS, IDX_MINOR), jnp.int32),
        pltpu.VMEM((CHUNK_TOK, EMB), jnp.float32),
        pltpu.VMEM((EMB // 8, CHUNK_TOK // 128, 8, 128), jnp.float32),
        pltpu.SemaphoreType.DMA,
    ],
    compiler_params=pltpu.CompilerParams(
        use_tc_tiling_on_sc=False, needs_layout_passes=False
    ),
)
def _emb_lookup(idx_hbm, table_hbm, out_hbm, idx_v, rows_v, trans_v, sem):
    wid = lax.axis_index("s") * NC + lax.axis_index("c")
    lane_iota = lax.iota(jnp.int32, 16)

    def chunk_body(ci, carry):
        c = wid * CHUNKS_PER_W + ci
        l = c // CHUNKS_PER_L
        bt0 = (c % CHUNKS_PER_L) * (CHUNK_TOK // 128)
        pltpu.sync_copy(idx_hbm.at[pl.ds(c * CHUNK_IROWS, CHUNK_IROWS)], idx_v)
        for j in range(CHUNK_IROWS):
            pltpu.async_copy(
                table_hbm.at[idx_v.at[j]],
                rows_v.at[pl.ds(j * IDX_MINOR, IDX_MINOR)],
                sem,
            )
        for j in range(CHUNK_IROWS):
            pltpu.make_async_copy(
                table_hbm.at[idx_v.at[j]],
                rows_v.at[pl.ds(j * IDX_MINOR, IDX_MINOR)],
                sem,
            ).wait()

        @plsc.parallel_loop(0, CHUNK_TOK // 16)
        def trans_body(gi):
            row_ids = gi * 16 + lane_iota
            btl = gi // 8
            g = gi % 8
            for et in range(EMB // 8):
                for es in range(8):
                    e = et * 8 + es
                    col_ids = jnp.full((16,), e, jnp.int32)
                    vals = plsc.load_gather(rows_v, [row_ids, col_ids])
                    trans_v[et, btl, es, pl.ds(g * 16, 16)] = vals * SCALE
        pltpu.sync_copy(
            trans_v,
            out_hbm.at[l, :, pl.ds(bt0, CHUNK_TOK // 128)],
        )
        return carry

    lax.fori_loop(0, CHUNKS_PER_W, chunk_body, 0)


def kernel(tokens, W):
    idx = jnp.reshape(tokens.T.astype(jnp.int32), (N // IDX_MINOR, IDX_MINOR))
    out5 = _emb_lookup(idx, W)
    return jnp.transpose(out5, (2, 4, 0, 1, 3)).reshape(B, L, EMB)


# speedup vs baseline: 1.4486x; 1.0650x over previous
import functools
import math

import jax
import jax.numpy as jnp
from jax import lax
from jax.experimental import pallas as pl
from jax.experimental.pallas import tpu as pltpu
from jax.experimental.pallas import tpu_sc as plsc

VOCAB = 1_000_000
EMB = 32
B = 4096
L = 200
N = B * L

NC = 2
NS = 16
NW = NC * NS

IDX_MINOR = 128
CHUNK_TOK = 512
CHUNK_IROWS = CHUNK_TOK // IDX_MINOR
CHUNK_BT = CHUNK_TOK // 128
CHUNKS_PER_L = B // CHUNK_TOK
N_CHUNKS_TOTAL = N // CHUNK_TOK
CHUNKS_PER_W = N_CHUNKS_TOTAL // NW

SCALE = math.sqrt(float(EMB))

_mesh = plsc.VectorSubcoreMesh(
    core_axis_name="c", subcore_axis_name="s", num_cores=NC, num_subcores=NS
)


@functools.partial(
    pl.kernel,
    out_type=jax.ShapeDtypeStruct((L, EMB // 8, B // 128, 8, 128), jnp.float32),
    mesh=_mesh,
    scratch_types=[
        pltpu.VMEM((CHUNK_IROWS, IDX_MINOR), jnp.int32),
        pltpu.VMEM((CHUNK_IROWS, IDX_MINOR), jnp.int32),
        pltpu.VMEM((CHUNK_TOK, EMB), jnp.float32),
        pltpu.VMEM((CHUNK_TOK, EMB), jnp.float32),
        pltpu.VMEM((EMB // 8, CHUNK_BT, 8, 128), jnp.float32),
        pltpu.VMEM((EMB // 8, CHUNK_BT, 8, 128), jnp.float32),
        pltpu.SemaphoreType.DMA,
        pltpu.SemaphoreType.DMA,
        pltpu.SemaphoreType.DMA,
        pltpu.SemaphoreType.DMA,
    ],
    compiler_params=pltpu.CompilerParams(
        use_tc_tiling_on_sc=False, needs_layout_passes=False
    ),
)
def _emb_lookup(
    idx_hbm, table_hbm, out_hbm,
    idx0, idx1, rows0, rows1, trans0, trans1,
    gsem0, gsem1, wsem0, wsem1,
):
    wid = lax.axis_index("s") * NC + lax.axis_index("c")
    base_c = wid * CHUNKS_PER_W
    lane_iota = lax.iota(jnp.int32, 16)
    bufs = (
        (idx0, rows0, trans0, gsem0, wsem0),
        (idx1, rows1, trans1, gsem1, wsem1),
    )

    def load_and_fire(ci, b):
        idxb, rowsb, _, gsemb, _ = bufs[b]
        c = base_c + ci
        pltpu.sync_copy(idx_hbm.at[pl.ds(c * CHUNK_IROWS, CHUNK_IROWS)], idxb)
        for j in range(CHUNK_IROWS):
            pltpu.async_copy(
                table_hbm.at[idxb.at[j]],
                rowsb.at[pl.ds(j * IDX_MINOR, IDX_MINOR)],
                gsemb,
            )

    def process(ci, b):
        idxb, rowsb, transb, gsemb, wsemb = bufs[b]
        c = base_c + ci
        l = c // CHUNKS_PER_L
        bt0 = (c % CHUNKS_PER_L) * CHUNK_BT
        out_win = out_hbm.at[l, :, pl.ds(bt0, CHUNK_BT)]
        for j in range(CHUNK_IROWS):
            pltpu.make_async_copy(
                table_hbm.at[idxb.at[j]],
                rowsb.at[pl.ds(j * IDX_MINOR, IDX_MINOR)],
                gsemb,
            ).wait()

        @pl.when(ci >= 2)
        def _():
            pltpu.make_async_copy(transb, out_win, wsemb).wait()

        @plsc.parallel_loop(0, CHUNK_TOK // 16)
        def trans_body(gi):
            row_ids = gi * 16 + lane_iota
            btl = gi // 8
            g = gi % 8
            for et in range(EMB // 8):
                for es in range(8):
                    e = et * 8 + es
                    col_ids = jnp.full((16,), e, jnp.int32)
                    vals = plsc.load_gather(rowsb, [row_ids, col_ids])
                    transb[et, btl, es, pl.ds(g * 16, 16)] = vals * SCALE

        pltpu.async_copy(transb, out_win, wsemb)

    load_and_fire(0, 0)

    @pl.loop(0, CHUNKS_PER_W, step=2)
    def chunk_loop(cb):
        for b in range(2):
            ci = cb + b

            @pl.when(ci + 1 < CHUNKS_PER_W)
            def _():
                load_and_fire(ci + 1, 1 - b)

            process(ci, b)

    pltpu.make_async_copy(trans0, out_hbm.at[0, :, pl.ds(0, CHUNK_BT)], wsem0).wait()
    pltpu.make_async_copy(trans1, out_hbm.at[0, :, pl.ds(0, CHUNK_BT)], wsem1).wait()


def kernel(tokens, W):
    idx = jnp.reshape(tokens.T.astype(jnp.int32), (N // IDX_MINOR, IDX_MINOR))
    out5 = _emb_lookup(idx, W)
    return jnp.transpose(out5, (2, 4, 0, 1, 3)).reshape(B, L, EMB)


# speedup vs baseline: 2.0378x; 1.4067x over previous
import functools
import math

import jax
import jax.numpy as jnp
from jax import lax
from jax.experimental import pallas as pl
from jax.experimental.pallas import tpu as pltpu
from jax.experimental.pallas import tpu_sc as plsc

VOCAB = 1_000_000
EMB = 32
B = 4096
L = 200
N = B * L

NC = 2
NS = 16
NW = NC * NS

IDX_MINOR = 128
CHUNK_TOK = 512
CHUNK_IROWS = CHUNK_TOK // IDX_MINOR
CHUNK_BT = CHUNK_TOK // 128
TRANS_MINOR = 129
CHUNKS_PER_L = B // CHUNK_TOK
N_CHUNKS_TOTAL = N // CHUNK_TOK
CHUNKS_PER_W = N_CHUNKS_TOTAL // NW

SCALE = math.sqrt(float(EMB))

_mesh = plsc.VectorSubcoreMesh(
    core_axis_name="c", subcore_axis_name="s", num_cores=NC, num_subcores=NS
)


@functools.partial(
    pl.kernel,
    out_type=jax.ShapeDtypeStruct((L, EMB // 8, B // 128, 8, 128), jnp.float32),
    mesh=_mesh,
    scratch_types=[
        pltpu.VMEM((CHUNK_IROWS, IDX_MINOR), jnp.int32),
        pltpu.VMEM((CHUNK_IROWS, IDX_MINOR), jnp.int32),
        pltpu.VMEM((CHUNK_TOK, EMB), jnp.float32),
        pltpu.VMEM((CHUNK_TOK, EMB), jnp.float32),
        pltpu.VMEM((EMB // 8, CHUNK_BT, 8, TRANS_MINOR), jnp.float32),
        pltpu.VMEM((EMB // 8, CHUNK_BT, 8, TRANS_MINOR), jnp.float32),
        pltpu.SemaphoreType.DMA,
        pltpu.SemaphoreType.DMA,
        pltpu.SemaphoreType.DMA,
        pltpu.SemaphoreType.DMA,
    ],
    compiler_params=pltpu.CompilerParams(
        use_tc_tiling_on_sc=False, needs_layout_passes=False
    ),
)
def _emb_lookup(
    idx_hbm, table_hbm, out_hbm,
    idx0, idx1, rows0, rows1, trans0, trans1,
    gsem0, gsem1, wsem0, wsem1,
):
    wid = lax.axis_index("s") * NC + lax.axis_index("c")
    base_c = wid * CHUNKS_PER_W
    lane_iota = lax.iota(jnp.int32, 16)
    zeros16 = jnp.zeros((16,), jnp.int32)
    et_lo = lane_iota >> 3
    et_hi = (lane_iota + 16) >> 3
    es_vec = lane_iota & 7
    bufs = (
        (idx0, rows0, trans0, gsem0, wsem0),
        (idx1, rows1, trans1, gsem1, wsem1),
    )

    def load_and_fire(ci, b):
        idxb, rowsb, _, gsemb, _ = bufs[b]
        c = base_c + ci
        pltpu.sync_copy(idx_hbm.at[pl.ds(c * CHUNK_IROWS, CHUNK_IROWS)], idxb)
        for j in range(CHUNK_IROWS):
            pltpu.async_copy(
                table_hbm.at[idxb.at[j]],
                rowsb.at[pl.ds(j * IDX_MINOR, IDX_MINOR)],
                gsemb,
            )

    def process(ci, b):
        idxb, rowsb, transb, gsemb, wsemb = bufs[b]
        c = base_c + ci
        l = c // CHUNKS_PER_L
        bt0 = (c % CHUNKS_PER_L) * CHUNK_BT
        out_win = out_hbm.at[l, :, pl.ds(bt0, CHUNK_BT)]
        trans_win = transb.at[:, :, :, pl.ds(0, 128)]
        for j in range(CHUNK_IROWS):
            pltpu.make_async_copy(
                table_hbm.at[idxb.at[j]],
                rowsb.at[pl.ds(j * IDX_MINOR, IDX_MINOR)],
                gsemb,
            ).wait()

        @pl.when(ci >= 2)
        def _():
            pltpu.make_async_copy(trans_win, out_win, wsemb).wait()

        @plsc.parallel_loop(0, CHUNK_TOK, unroll=4)
        def trans_body(t):
            btl_vec = zeros16 + (t >> 7)
            bl_vec = zeros16 + (t & 127)
            v0 = rowsb[t, pl.ds(0, 16)] * SCALE
            v1 = rowsb[t, pl.ds(16, 16)] * SCALE
            plsc.store_scatter(transb, [et_lo, btl_vec, es_vec, bl_vec], v0)
            plsc.store_scatter(transb, [et_hi, btl_vec, es_vec, bl_vec], v1)

        pltpu.async_copy(trans_win, out_win, wsemb)

    load_and_fire(0, 0)

    @pl.loop(0, CHUNKS_PER_W, step=2)
    def chunk_loop(cb):
        for b in range(2):
            ci = cb + b

            @pl.when(ci + 1 < CHUNKS_PER_W)
            def _():
                load_and_fire(ci + 1, 1 - b)

            process(ci, b)

    pltpu.make_async_copy(
        trans0.at[:, :, :, pl.ds(0, 128)],
        out_hbm.at[0, :, pl.ds(0, CHUNK_BT)], wsem0).wait()
    pltpu.make_async_copy(
        trans1.at[:, :, :, pl.ds(0, 128)],
        out_hbm.at[0, :, pl.ds(0, CHUNK_BT)], wsem1).wait()


def kernel(tokens, W):
    idx = jnp.reshape(tokens.T.astype(jnp.int32), (N // IDX_MINOR, IDX_MINOR))
    out5 = _emb_lookup(idx, W)
    return jnp.transpose(out5, (2, 4, 0, 1, 3)).reshape(B, L, EMB)


# speedup vs baseline: 2.2567x; 1.1074x over previous
import functools
import math

import jax
import jax.numpy as jnp
from jax import lax
from jax.experimental import pallas as pl
from jax.experimental.pallas import tpu as pltpu
from jax.experimental.pallas import tpu_sc as plsc

VOCAB = 1_000_000
EMB = 32
B = 4096
L = 200
N = B * L

NC = 2
NS = 16
NW = NC * NS

IDX_MINOR = 128
CHUNK_TOK = 512
CHUNK_IROWS = CHUNK_TOK // IDX_MINOR
CHUNK_BT = CHUNK_TOK // 128
TRANS_MINOR = 129
CHUNKS_PER_L = B // CHUNK_TOK
N_CHUNKS_TOTAL = N // CHUNK_TOK
CHUNKS_PER_W = N_CHUNKS_TOTAL // NW

SCALE = math.sqrt(float(EMB))

_mesh = plsc.VectorSubcoreMesh(
    core_axis_name="c", subcore_axis_name="s", num_cores=NC, num_subcores=NS
)


@functools.partial(
    pl.kernel,
    out_type=jax.ShapeDtypeStruct((L, EMB // 8, B // 128, 8, 128), jnp.float32),
    mesh=_mesh,
    scratch_types=[
        pltpu.VMEM((CHUNK_IROWS, IDX_MINOR), jnp.int32),
        pltpu.VMEM((CHUNK_IROWS, IDX_MINOR), jnp.int32),
        pltpu.VMEM((CHUNK_TOK, EMB), jnp.float32),
        pltpu.VMEM((CHUNK_TOK, EMB), jnp.float32),
        pltpu.VMEM((EMB // 8, CHUNK_BT, 8, TRANS_MINOR), jnp.float32),
        pltpu.VMEM((EMB // 8, CHUNK_BT, 8, TRANS_MINOR), jnp.float32),
        pltpu.SemaphoreType.DMA,
        pltpu.SemaphoreType.DMA,
        pltpu.SemaphoreType.DMA,
        pltpu.SemaphoreType.DMA,
    ],
    compiler_params=pltpu.CompilerParams(
        use_tc_tiling_on_sc=False, needs_layout_passes=False
    ),
)
def _emb_lookup(
    idx_hbm, table_hbm, out_hbm,
    idx0, idx1, rows0, rows1, trans0, trans1,
    gsem0, gsem1, wsem0, wsem1,
):
    wid = lax.axis_index("s") * NC + lax.axis_index("c")
    base_c = wid * CHUNKS_PER_W
    lane_iota = lax.iota(jnp.int32, 16)
    zeros16 = jnp.zeros((16,), jnp.int32)
    et_lo = lane_iota >> 3
    et_hi = (lane_iota + 16) >> 3
    es_vec = lane_iota & 7
    bufs = (
        (idx0, rows0, trans0, gsem0, wsem0),
        (idx1, rows1, trans1, gsem1, wsem1),
    )

    def load_and_fire(ci, b):
        idxb, rowsb, _, gsemb, _ = bufs[b]
        c = base_c + ci
        pltpu.sync_copy(idx_hbm.at[pl.ds(c * CHUNK_IROWS, CHUNK_IROWS)], idxb)
        for j in range(CHUNK_IROWS):
            pltpu.async_copy(
                table_hbm.at[idxb.at[j]],
                rowsb.at[pl.ds(j * IDX_MINOR, IDX_MINOR)],
                gsemb,
            )

    def process(ci, b):
        idxb, rowsb, transb, gsemb, wsemb = bufs[b]
        c = base_c + ci
        l = c // CHUNKS_PER_L
        bt0 = (c % CHUNKS_PER_L) * CHUNK_BT
        out_win = out_hbm.at[l, :, pl.ds(bt0, CHUNK_BT)]
        trans_win = transb.at[:, :, :, pl.ds(0, 128)]
        for j in range(CHUNK_IROWS):
            pltpu.make_async_copy(
                table_hbm.at[idxb.at[j]],
                rowsb.at[pl.ds(j * IDX_MINOR, IDX_MINOR)],
                gsemb,
            ).wait()

        @pl.when(ci >= 2)
        def _():
            pltpu.make_async_copy(trans_win, out_win, wsemb).wait()

        @plsc.parallel_loop(0, CHUNK_TOK, unroll=4)
        def trans_body(t):
            btl_vec = zeros16 + (t >> 7)
            bl_vec = zeros16 + (t & 127)
            v0 = rowsb[t, pl.ds(0, 16)]
            v1 = rowsb[t, pl.ds(16, 16)]
            plsc.store_scatter(transb, [et_lo, btl_vec, es_vec, bl_vec], v0)
            plsc.store_scatter(transb, [et_hi, btl_vec, es_vec, bl_vec], v1)

        pltpu.async_copy(trans_win, out_win, wsemb)

    load_and_fire(0, 0)

    @pl.loop(0, CHUNKS_PER_W, step=2)
    def chunk_loop(cb):
        for b in range(2):
            ci = cb + b

            @pl.when(ci + 1 < CHUNKS_PER_W)
            def _():
                load_and_fire(ci + 1, 1 - b)

            process(ci, b)

    pltpu.make_async_copy(
        trans0.at[:, :, :, pl.ds(0, 128)],
        out_hbm.at[0, :, pl.ds(0, CHUNK_BT)], wsem0).wait()
    pltpu.make_async_copy(
        trans1.at[:, :, :, pl.ds(0, 128)],
        out_hbm.at[0, :, pl.ds(0, CHUNK_BT)], wsem1).wait()


_VBLK = 2048


def _w_prep_body(wt_ref, out_ref):
    out_ref[:, 0:EMB] = wt_ref[...].T * SCALE


def _w_prep(wt):
    return pl.pallas_call(
        _w_prep_body,
        grid=((VOCAB + _VBLK - 1) // _VBLK,),
        in_specs=[pl.BlockSpec((EMB, _VBLK), lambda i: (0, i))],
        out_specs=pl.BlockSpec((_VBLK, 128), lambda i: (i, 0)),
        out_shape=jax.ShapeDtypeStruct((VOCAB, 128), jnp.float32),
    )(wt)


def kernel(tokens, W):
    idx = jnp.reshape(tokens.T.astype(jnp.int32) * 4, (N // IDX_MINOR, IDX_MINOR))
    table = jnp.reshape(_w_prep(W.T), (4 * VOCAB, EMB))
    out5 = _emb_lookup(idx, table)
    return jnp.transpose(out5, (2, 4, 0, 1, 3)).reshape(B, L, EMB)
